# Initial kernel scaffold; baseline (speedup 1.0000x reference)
import functools

import jax
import jax.numpy as jnp
from jax import lax
from jax.experimental import pallas as pl
from jax.experimental.pallas import tpu as pltpu
from jax.experimental.pallas import tpu_sc as plsc

N = 10000
E = 160000
D = 256
DH = 128
NSUB = 16
BLK = 128
NB = 79
EPAD = NSUB * NB * BLK
ACC_ROWS = 10240
ZROWS = ACC_ROWS // NSUB
OROWS = N // NSUB


def _sc_aggregate(feats_lo, feats_hi, src_idx, dst_idx):
  mesh = plsc.VectorSubcoreMesh(core_axis_name="c", subcore_axis_name="s")
  f32 = jnp.float32

  @functools.partial(
      pl.kernel,
      out_type=(
          jax.ShapeDtypeStruct((N, DH), f32),
          jax.ShapeDtypeStruct((N, DH), f32),
          jax.ShapeDtypeStruct((N, 16), f32),
          jax.ShapeDtypeStruct((N, 16), f32),
      ),
      mesh=mesh,
      scratch_types=[
          pltpu.VMEM((BLK, DH), f32),
          pltpu.VMEM((BLK, DH), f32),
          pltpu.VMEM((BLK, 16), f32),
          pltpu.VMEM((BLK, 16), f32),
          pltpu.VMEM((BLK,), jnp.int32),
          pltpu.VMEM((BLK,), jnp.int32),
          pltpu.VMEM_SHARED((ACC_ROWS, DH), f32),
          pltpu.VMEM_SHARED((ACC_ROWS, 16), f32),
          pltpu.SemaphoreType.DMA,
      ],
  )
  def k(lo_hbm, hi_hbm, src_hbm, dst_hbm,
        out_lo, out_hi, out_d0, out_d1,
        rows_v, zero_v, ones_v, zero16_v, sidx_v, didx_v,
        acc_sum, acc_deg, sem):
    c = lax.axis_index("c")
    s = lax.axis_index("s")

    @pl.loop(0, BLK)
    def _(r):
      ones_v[r, pl.ds(0, 16)] = jnp.ones((16,), f32)
      zero16_v[r, pl.ds(0, 16)] = jnp.zeros((16,), f32)

      @pl.loop(0, DH, step=16)
      def _(cc):
        zero_v[r, pl.ds(cc, 16)] = jnp.zeros((16,), f32)

    zbase = s * ZROWS

    @pl.loop(0, ZROWS, step=BLK)
    def _(r):
      pltpu.sync_copy(zero_v, acc_sum.at[pl.ds(zbase + r, BLK)])
      pltpu.sync_copy(zero16_v, acc_deg.at[pl.ds(zbase + r, BLK)])

    plsc.subcore_barrier()

    def run_half(feats_hbm, parity):
      @pl.loop(0, NB)
      def _(j):
        pltpu.sync_copy(src_hbm.at[s, j], sidx_v)
        pltpu.sync_copy(dst_hbm.at[s, j], didx_v)
        pltpu.async_copy(feats_hbm.at[sidx_v], rows_v, sem).wait()
        pltpu.sync_copy(rows_v, acc_sum.at[didx_v], add=True)

        @pl.when(j % 2 == parity)
        def _():
          pltpu.sync_copy(ones_v, acc_deg.at[didx_v], add=True)

    @pl.when(c == 0)
    def _():
      run_half(lo_hbm, 0)

    @pl.when(c == 1)
    def _():
      run_half(hi_hbm, 1)

    plsc.subcore_barrier()

    obase = s * OROWS

    @pl.when(c == 0)
    def _():
      pltpu.sync_copy(acc_sum.at[pl.ds(obase, OROWS)],
                      out_lo.at[pl.ds(obase, OROWS)])
      pltpu.sync_copy(acc_deg.at[pl.ds(obase, OROWS)],
                      out_d0.at[pl.ds(obase, OROWS)])

    @pl.when(c == 1)
    def _():
      pltpu.sync_copy(acc_sum.at[pl.ds(obase, OROWS)],
                      out_hi.at[pl.ds(obase, OROWS)])
      pltpu.sync_copy(acc_deg.at[pl.ds(obase, OROWS)],
                      out_d1.at[pl.ds(obase, OROWS)])

  return k(feats_lo, feats_hi, src_idx, dst_idx)


_TC_BLK = 400


def _tc_body(feats_ref, lo_ref, hi_ref, d0_ref, d1_ref,
             ws_ref, wn_ref, b_ref, out_ref):
  deg = d0_ref[:, 0:1] + d1_ref[:, 0:1]
  deg = jnp.maximum(deg, 1.0)
  h = jnp.concatenate([lo_ref[...], hi_ref[...]], axis=1) / deg
  acc = jnp.dot(feats_ref[...], ws_ref[...], preferred_element_type=jnp.float32)
  acc = acc + jnp.dot(h, wn_ref[...], preferred_element_type=jnp.float32)
  out_ref[...] = jnp.maximum(acc + b_ref[...], 0.0)


def _tc_combine(feats, sum_lo, sum_hi, deg0, deg1, W_self, W_neigh, bias):
  grid = (N // _TC_BLK,)
  return pl.pallas_call(
      _tc_body,
      grid=grid,
      in_specs=[
          pl.BlockSpec((_TC_BLK, D), lambda i: (i, 0)),
          pl.BlockSpec((_TC_BLK, DH), lambda i: (i, 0)),
          pl.BlockSpec((_TC_BLK, DH), lambda i: (i, 0)),
          pl.BlockSpec((_TC_BLK, 16), lambda i: (i, 0)),
          pl.BlockSpec((_TC_BLK, 16), lambda i: (i, 0)),
          pl.BlockSpec((D, D), lambda i: (0, 0)),
          pl.BlockSpec((D, D), lambda i: (0, 0)),
          pl.BlockSpec((1, D), lambda i: (0, 0)),
      ],
      out_specs=pl.BlockSpec((_TC_BLK, D), lambda i: (i, 0)),
      out_shape=jax.ShapeDtypeStruct((N, D), jnp.float32),
  )(feats, sum_lo, sum_hi, deg0, deg1, W_self, W_neigh, bias)


def kernel(feats, edge_index, W_self, W_neigh, bias):
  src = edge_index[0].astype(jnp.int32)
  dst = edge_index[1].astype(jnp.int32)
  npad = EPAD - E
  src_p = jnp.concatenate([src, jnp.zeros((npad,), jnp.int32)])
  dst_p = jnp.concatenate([dst, jnp.full((npad,), N, jnp.int32)])
  src_idx = src_p.reshape(NSUB, NB, BLK)
  dst_idx = dst_p.reshape(NSUB, NB, BLK)

  feats_lo = feats[:, :DH]
  feats_hi = feats[:, DH:]

  sum_lo, sum_hi, deg0, deg1 = _sc_aggregate(feats_lo, feats_hi,
                                             src_idx, dst_idx)
  return _tc_combine(feats, sum_lo, sum_hi, deg0, deg1,
                     W_self, W_neigh, bias.reshape(1, D))


# speedup vs baseline: 1.0675x; 1.0675x over previous
import functools

import jax
import jax.numpy as jnp
from jax import lax
from jax.experimental import pallas as pl
from jax.experimental.pallas import tpu as pltpu
from jax.experimental.pallas import tpu_sc as plsc

N = 10000
E = 160000
D = 256
DH = 128
NSUB = 16
BLK = 128
NB = 80
EPAD = NSUB * NB * BLK

NPASS = 3
STRIDE = 3336
R_ACC = 3456
DUMP = 3400
ZR = R_ACC // NSUB
CR = 208
OUT_ROWS = NPASS * STRIDE


def _sc_aggregate(feats_lo, feats_hi, src_idx, dst_loc, zeros_z, ones_b):
  mesh = plsc.VectorSubcoreMesh(core_axis_name="c", subcore_axis_name="s")
  f32 = jnp.float32

  @functools.partial(
      pl.kernel,
      out_type=(
          jax.ShapeDtypeStruct((OUT_ROWS, DH), f32),
          jax.ShapeDtypeStruct((OUT_ROWS, DH), f32),
          jax.ShapeDtypeStruct((OUT_ROWS, DH), f32),
          jax.ShapeDtypeStruct((OUT_ROWS, DH), f32),
      ),
      mesh=mesh,
      scratch_types=[
          pltpu.VMEM((BLK, DH), f32),
          pltpu.VMEM((ZR, DH), f32),
          pltpu.VMEM((BLK, DH), f32),
          pltpu.VMEM((NB, BLK), jnp.int32),
          pltpu.VMEM((NB, BLK), jnp.int32),
          pltpu.VMEM_SHARED((R_ACC, DH), f32),
          pltpu.SemaphoreType.DMA,
      ],
  )
  def k(lo_hbm, hi_hbm, src_hbm, dloc_hbm, zz_hbm, ones_hbm,
        out_lo, out_hi, out_d0, out_d1,
        rows_v, zero_v, ones_v, sidx_v, dloc_v, acc, sem):
    c = lax.axis_index("c")
    s = lax.axis_index("s")

    pltpu.sync_copy(zz_hbm, zero_v)
    pltpu.sync_copy(ones_hbm, ones_v)
    pltpu.sync_copy(src_hbm.at[s], sidx_v)

    def start_pass(p):
      pltpu.sync_copy(dloc_hbm.at[p, s], dloc_v)
      pltpu.sync_copy(zero_v, acc.at[pl.ds(s * ZR, ZR)])
      plsc.subcore_barrier()

    def end_pass(out_hbm, p):
      plsc.subcore_barrier()
      base = p * STRIDE
      pltpu.sync_copy(acc.at[pl.ds(s * CR, CR)],
                      out_hbm.at[pl.ds(base + s * CR, CR)])

      @pl.when(s == NSUB - 1)
      def _():
        pltpu.sync_copy(acc.at[pl.ds(NSUB * CR, STRIDE - NSUB * CR)],
                        out_hbm.at[pl.ds(base + NSUB * CR,
                                         STRIDE - NSUB * CR)])

      plsc.subcore_barrier()

    def sum_pass(feats_hbm, out_hbm, p):
      start_pass(p)

      @pl.loop(0, NB)
      def _(j):
        pltpu.async_copy(feats_hbm.at[sidx_v.at[j]], rows_v, sem).wait()
        pltpu.sync_copy(rows_v, acc.at[dloc_v.at[j]], add=True)

      end_pass(out_hbm, p)

    def deg_pass(out_hbm, parity, p):
      start_pass(p)

      @pl.loop(0, NB)
      def _(j):
        @pl.when(j % 2 == parity)
        def _():
          pltpu.sync_copy(ones_v, acc.at[dloc_v.at[j]], add=True)

      end_pass(out_hbm, p)

    @pl.when(c == 0)
    def _():
      for p in range(NPASS):
        sum_pass(lo_hbm, out_lo, p)
      for p in range(NPASS):
        deg_pass(out_d0, 0, p)

    @pl.when(c == 1)
    def _():
      for p in range(NPASS):
        sum_pass(hi_hbm, out_hi, p)
      for p in range(NPASS):
        deg_pass(out_d1, 1, p)

  return k(feats_lo, feats_hi, src_idx, dst_loc, zeros_z, ones_b)


_TC_BLK = 400


def _tc_body(feats_ref, lo_ref, hi_ref, d0_ref, d1_ref,
             ws_ref, wn_ref, b_ref, out_ref):
  deg = d0_ref[:, 0:1] + d1_ref[:, 0:1]
  deg = jnp.maximum(deg, 1.0)
  h = jnp.concatenate([lo_ref[...], hi_ref[...]], axis=1) / deg
  acc = jnp.dot(feats_ref[...], ws_ref[...], preferred_element_type=jnp.float32)
  acc = acc + jnp.dot(h, wn_ref[...], preferred_element_type=jnp.float32)
  out_ref[...] = jnp.maximum(acc + b_ref[...], 0.0)


def _tc_combine(feats, sum_lo, sum_hi, deg0, deg1, W_self, W_neigh, bias):
  grid = (N // _TC_BLK,)
  return pl.pallas_call(
      _tc_body,
      grid=grid,
      in_specs=[
          pl.BlockSpec((_TC_BLK, D), lambda i: (i, 0)),
          pl.BlockSpec((_TC_BLK, DH), lambda i: (i, 0)),
          pl.BlockSpec((_TC_BLK, DH), lambda i: (i, 0)),
          pl.BlockSpec((_TC_BLK, DH), lambda i: (i, 0)),
          pl.BlockSpec((_TC_BLK, DH), lambda i: (i, 0)),
          pl.BlockSpec((D, D), lambda i: (0, 0)),
          pl.BlockSpec((D, D), lambda i: (0, 0)),
          pl.BlockSpec((1, D), lambda i: (0, 0)),
      ],
      out_specs=pl.BlockSpec((_TC_BLK, D), lambda i: (i, 0)),
      out_shape=jax.ShapeDtypeStruct((N, D), jnp.float32),
  )(feats, sum_lo, sum_hi, deg0, deg1, W_self, W_neigh, bias)


def kernel(feats, edge_index, W_self, W_neigh, bias):
  src = edge_index[0].astype(jnp.int32)
  dst = edge_index[1].astype(jnp.int32)
  npad = EPAD - E
  src_p = jnp.concatenate([src, jnp.zeros((npad,), jnp.int32)])
  dst_p = jnp.concatenate([dst, jnp.full((npad,), N, jnp.int32)])
  src_idx = src_p.reshape(NSUB, NB, BLK)

  locs = []
  for p in range(NPASS):
    t = dst_p - p * STRIDE
    locs.append(jnp.where((t >= 0) & (t < STRIDE), t, DUMP))
  dst_loc = jnp.stack(locs).reshape(NPASS, NSUB, NB, BLK)

  zeros_z = jnp.zeros((ZR, DH), jnp.float32)
  ones_b = jnp.ones((BLK, DH), jnp.float32)

  feats_lo = feats[:, :DH]
  feats_hi = feats[:, DH:]

  sum_lo, sum_hi, deg0, deg1 = _sc_aggregate(
      feats_lo, feats_hi, src_idx, dst_loc, zeros_z, ones_b)
  return _tc_combine(feats, sum_lo, sum_hi, deg0, deg1,
                     W_self, W_neigh, bias.reshape(1, D))


# speedup vs baseline: 1.1843x; 1.1094x over previous
import functools

import jax
import jax.numpy as jnp
from jax import lax
from jax.experimental import pallas as pl
from jax.experimental.pallas import tpu as pltpu
from jax.experimental.pallas import tpu_sc as plsc

N = 10000
E = 160000
D = 256
DH = 128
NSUB = 16
BLK = 128
NB = 80
EPAD = NSUB * NB * BLK

NPASS = 3
STRIDE = 3336
R_ACC = 3456
DUMP = 3400
ZR = R_ACC // NSUB
CR = 208
OUT_ROWS = NPASS * STRIDE


def _sc_aggregate(feats_lo, feats_hi, src_idx, dst_loc, zeros_z, ones_b):
  mesh = plsc.VectorSubcoreMesh(core_axis_name="c", subcore_axis_name="s")
  f32 = jnp.float32

  @functools.partial(
      pl.kernel,
      out_type=(
          jax.ShapeDtypeStruct((OUT_ROWS, DH), f32),
          jax.ShapeDtypeStruct((OUT_ROWS, DH), f32),
          jax.ShapeDtypeStruct((OUT_ROWS, DH), f32),
          jax.ShapeDtypeStruct((OUT_ROWS, DH), f32),
      ),
      mesh=mesh,
      scratch_types=[
          pltpu.VMEM((BLK, DH), f32),
          pltpu.VMEM((BLK, DH), f32),
          pltpu.VMEM((ZR, DH), f32),
          pltpu.VMEM((BLK, DH), f32),
          pltpu.VMEM((NB, BLK), jnp.int32),
          pltpu.VMEM((NB, BLK), jnp.int32),
          pltpu.VMEM_SHARED((R_ACC, DH), f32),
          pltpu.SemaphoreType.DMA,
          pltpu.SemaphoreType.DMA,
          pltpu.SemaphoreType.DMA,
          pltpu.SemaphoreType.DMA,
      ],
  )
  def k(lo_hbm, hi_hbm, src_hbm, dloc_hbm, zz_hbm, ones_hbm,
        out_lo, out_hi, out_d0, out_d1,
        rows_a, rows_b, zero_v, ones_v, sidx_v, dloc_v, acc,
        sem_ga, sem_gb, sem_sa, sem_sb):
    c = lax.axis_index("c")
    s = lax.axis_index("s")

    pltpu.sync_copy(zz_hbm, zero_v)
    pltpu.sync_copy(ones_hbm, ones_v)
    pltpu.sync_copy(src_hbm.at[s], sidx_v)

    def start_pass(p):
      pltpu.sync_copy(dloc_hbm.at[p, s], dloc_v)
      pltpu.sync_copy(zero_v, acc.at[pl.ds(s * ZR, ZR)])
      plsc.subcore_barrier()

    def end_pass(out_hbm, p):
      plsc.subcore_barrier()
      base = p * STRIDE
      pltpu.sync_copy(acc.at[pl.ds(s * CR, CR)],
                      out_hbm.at[pl.ds(base + s * CR, CR)])

      @pl.when(s == NSUB - 1)
      def _():
        pltpu.sync_copy(acc.at[pl.ds(NSUB * CR, STRIDE - NSUB * CR)],
                        out_hbm.at[pl.ds(base + NSUB * CR,
                                         STRIDE - NSUB * CR)])

      plsc.subcore_barrier()

    def sum_pass(feats_hbm, out_hbm, p):
      start_pass(p)

      pltpu.async_copy(feats_hbm.at[sidx_v.at[0]], rows_a, sem_ga)
      pltpu.async_copy(feats_hbm.at[sidx_v.at[1]], rows_b, sem_gb)

      @pl.loop(0, NB, step=2)
      def _(jj):
        for (buf, sem_g, sem_s, off) in ((rows_a, sem_ga, sem_sa, 0),
                                         (rows_b, sem_gb, sem_sb, 1)):
          j = jj + off
          pltpu.make_async_copy(feats_hbm.at[sidx_v.at[j]], buf, sem_g).wait()
          pltpu.async_copy(buf, acc.at[dloc_v.at[j]], sem_s, add=True).wait()

          @pl.when(j + 2 < NB)
          def _():
            pltpu.async_copy(feats_hbm.at[sidx_v.at[j + 2]], buf, sem_g)

      end_pass(out_hbm, p)

    def deg_pass(out_hbm, parity, p):
      start_pass(p)

      @pl.loop(0, NB)
      def _(j):
        @pl.when(j % 2 == parity)
        def _():
          pltpu.sync_copy(ones_v, acc.at[dloc_v.at[j]], add=True)

      end_pass(out_hbm, p)

    @pl.when(c == 0)
    def _():
      for p in range(NPASS):
        sum_pass(lo_hbm, out_lo, p)
      for p in range(NPASS):
        deg_pass(out_d0, 0, p)

    @pl.when(c == 1)
    def _():
      for p in range(NPASS):
        sum_pass(hi_hbm, out_hi, p)
      for p in range(NPASS):
        deg_pass(out_d1, 1, p)

  return k(feats_lo, feats_hi, src_idx, dst_loc, zeros_z, ones_b)


_TC_BLK = 400


def _tc_body(feats_ref, lo_ref, hi_ref, d0_ref, d1_ref,
             ws_ref, wn_ref, b_ref, out_ref):
  deg = d0_ref[:, 0:1] + d1_ref[:, 0:1]
  deg = jnp.maximum(deg, 1.0)
  h = jnp.concatenate([lo_ref[...], hi_ref[...]], axis=1) / deg
  acc = jnp.dot(feats_ref[...], ws_ref[...], preferred_element_type=jnp.float32)
  acc = acc + jnp.dot(h, wn_ref[...], preferred_element_type=jnp.float32)
  out_ref[...] = jnp.maximum(acc + b_ref[...], 0.0)


def _tc_combine(feats, sum_lo, sum_hi, deg0, deg1, W_self, W_neigh, bias):
  grid = (N // _TC_BLK,)
  return pl.pallas_call(
      _tc_body,
      grid=grid,
      in_specs=[
          pl.BlockSpec((_TC_BLK, D), lambda i: (i, 0)),
          pl.BlockSpec((_TC_BLK, DH), lambda i: (i, 0)),
          pl.BlockSpec((_TC_BLK, DH), lambda i: (i, 0)),
          pl.BlockSpec((_TC_BLK, DH), lambda i: (i, 0)),
          pl.BlockSpec((_TC_BLK, DH), lambda i: (i, 0)),
          pl.BlockSpec((D, D), lambda i: (0, 0)),
          pl.BlockSpec((D, D), lambda i: (0, 0)),
          pl.BlockSpec((1, D), lambda i: (0, 0)),
      ],
      out_specs=pl.BlockSpec((_TC_BLK, D), lambda i: (i, 0)),
      out_shape=jax.ShapeDtypeStruct((N, D), jnp.float32),
  )(feats, sum_lo, sum_hi, deg0, deg1, W_self, W_neigh, bias)


def kernel(feats, edge_index, W_self, W_neigh, bias):
  src = edge_index[0].astype(jnp.int32)
  dst = edge_index[1].astype(jnp.int32)
  npad = EPAD - E
  src_p = jnp.concatenate([src, jnp.zeros((npad,), jnp.int32)])
  dst_p = jnp.concatenate([dst, jnp.full((npad,), N, jnp.int32)])
  src_idx = src_p.reshape(NSUB, NB, BLK)

  locs = []
  for p in range(NPASS):
    t = dst_p - p * STRIDE
    locs.append(jnp.where((t >= 0) & (t < STRIDE), t, DUMP))
  dst_loc = jnp.stack(locs).reshape(NPASS, NSUB, NB, BLK)

  zeros_z = jnp.zeros((ZR, DH), jnp.float32)
  ones_b = jnp.ones((BLK, DH), jnp.float32)

  feats_lo = feats[:, :DH]
  feats_hi = feats[:, DH:]

  sum_lo, sum_hi, deg0, deg1 = _sc_aggregate(
      feats_lo, feats_hi, src_idx, dst_loc, zeros_z, ones_b)
  return _tc_combine(feats, sum_lo, sum_hi, deg0, deg1,
                     W_self, W_neigh, bias.reshape(1, D))


# speedup vs baseline: 2.4911x; 2.1035x over previous
import dataclasses
import functools

import jax
import jax.numpy as jnp
from jax import lax
from jax.experimental import pallas as pl
from jax.experimental.pallas import tpu as pltpu
from jax.experimental.pallas import tpu_sc as plsc

N = 10000
E = 160000
D = 256
DH = 128
NSUB = 16
BLK = 128
NB = 80
EPAD = NSUB * NB * BLK
SLAB = NB * BLK
CAP = SLAB + BLK

NPASS = 3
STRIDE = 3336
R_ACC = 3456
DUMP = 3400
ZR = R_ACC // NSUB
CR = 208
OUT_ROWS = NPASS * STRIDE


def _sc_aggregate(feats_lo, feats_hi, src_idx, dst_loc, zeros_z, ones_b):
  mesh = plsc.VectorSubcoreMesh(core_axis_name="c", subcore_axis_name="s")
  f32 = jnp.float32
  i32 = jnp.int32

  @functools.partial(
      pl.kernel,
      out_type=(
          jax.ShapeDtypeStruct((OUT_ROWS, DH), f32),
          jax.ShapeDtypeStruct((OUT_ROWS, DH), f32),
          jax.ShapeDtypeStruct((OUT_ROWS, DH), f32),
          jax.ShapeDtypeStruct((OUT_ROWS, DH), f32),
      ),
      mesh=mesh,
      scratch_types=[
          pltpu.VMEM((BLK, DH), f32),
          pltpu.VMEM((BLK, DH), f32),
          pltpu.VMEM((8, DH), f32),
          pltpu.VMEM((BLK, DH), f32),
          pltpu.VMEM((NB, BLK), i32),
          pltpu.VMEM((NB, BLK), i32),
          pltpu.VMEM((CAP,), i32),
          pltpu.VMEM((CAP,), i32),
          pltpu.VMEM((CAP,), i32),
          pltpu.VMEM((BLK,), i32),
          pltpu.VMEM_SHARED((R_ACC, DH), f32),
          pltpu.SemaphoreType.DMA,
          pltpu.SemaphoreType.DMA,
          pltpu.SemaphoreType.DMA,
      ],
      compiler_params=dataclasses.replace(pltpu.CompilerParams(),
                                          needs_layout_passes=False),
  )
  def k(lo_hbm, hi_hbm, src_hbm, dloc_hbm, zz_hbm, ones_hbm,
        out_lo, out_hi, out_d0, out_d1,
        rows_a, rows_b, zero_v, ones_v, sidx_v, dloc_v,
        csrc, cdst, cdeg, dblk_v, acc,
        sem_ga, sem_gb, sem_s):
    c = lax.axis_index("c")
    s = lax.axis_index("s")

    pltpu.sync_copy(zz_hbm, zero_v)
    pltpu.sync_copy(ones_hbm, ones_v)
    pltpu.sync_copy(src_hbm.at[s], sidx_v)

    def zero_acc():
      @pl.loop(0, ZR, step=8)
      def _(r):
        pltpu.sync_copy(zero_v, acc.at[pl.ds(s * ZR + r, 8)])

    def copy_out(out_hbm, p):
      base = p * STRIDE
      pltpu.sync_copy(acc.at[pl.ds(s * CR, CR)],
                      out_hbm.at[pl.ds(base + s * CR, CR)])

      @pl.when(s == NSUB - 1)
      def _():
        pltpu.sync_copy(acc.at[pl.ds(NSUB * CR, STRIDE - NSUB * CR)],
                        out_hbm.at[pl.ds(base + NSUB * CR,
                                         STRIDE - NSUB * CR)])

    def compact(parity):
      def body(j, carry):
        cs, cd = carry
        pj = (j % 2) == parity
        for rr in range(BLK // 16):
          dv = dloc_v[j, pl.ds(rr * 16, 16)]
          sv = sidx_v[j, pl.ds(rr * 16, 16)]
          m = dv < STRIDE
          plsc.store_compressed(csrc.at[pl.ds(cs, 16)], sv, mask=m)
          plsc.store_compressed(cdst.at[pl.ds(cs, 16)], dv, mask=m)
          cs = cs + jnp.sum(m.astype(i32))
          md = m & pj
          plsc.store_compressed(cdeg.at[pl.ds(cd, 16)], dv, mask=md)
          cd = cd + jnp.sum(md.astype(i32))
        return (cs, cd)

      cs, cd = lax.fori_loop(0, NB, body, (jnp.int32(0), jnp.int32(0)))
      zeros16 = jnp.zeros((16,), i32)
      dump16 = jnp.full((16,), DUMP, i32)
      for t in range(BLK // 16):
        csrc[pl.ds(cs + t * 16, 16)] = zeros16
        cdst[pl.ds(cs + t * 16, 16)] = dump16
        cdeg[pl.ds(cd + t * 16, 16)] = dump16
      return (cs + 127) // 128, (cd + 127) // 128

    def sum_blocks(feats_hbm, nb_s):
      @pl.when(nb_s > 0)
      def _():
        pltpu.async_copy(feats_hbm.at[csrc.at[pl.ds(0, BLK)]],
                         rows_a, sem_ga)

      @pl.when(nb_s > 1)
      def _():
        pltpu.async_copy(feats_hbm.at[csrc.at[pl.ds(BLK, BLK)]],
                         rows_b, sem_gb)

      def body(bb, _):
        for (buf, sem_g, off) in ((rows_a, sem_ga, 0), (rows_b, sem_gb, 1)):
          b = bb * 2 + off

          @pl.when(b < nb_s)
          def _():
            pltpu.make_async_copy(
                feats_hbm.at[csrc.at[pl.ds(b * BLK, BLK)]],
                buf, sem_g).wait()
            for rr in range(BLK // 16):
              dblk_v[pl.ds(rr * 16, 16)] = cdst[pl.ds(b * BLK + rr * 16, 16)]
            pltpu.sync_copy(buf, acc.at[dblk_v], add=True)

            @pl.when(b + 2 < nb_s)
            def _():
              pltpu.async_copy(
                  feats_hbm.at[csrc.at[pl.ds((b + 2) * BLK, BLK)]],
                  buf, sem_g)
        return 0

      lax.fori_loop(0, (nb_s + 1) // 2, body, 0)

    def deg_blocks(nb_d):
      def body(b, _):
        for rr in range(BLK // 16):
          dblk_v[pl.ds(rr * 16, 16)] = cdeg[pl.ds(b * BLK + rr * 16, 16)]
        pltpu.sync_copy(ones_v, acc.at[dblk_v], add=True)
        return 0

      lax.fori_loop(0, nb_d, body, 0)

    def run_core(feats_hbm, out_sum, out_deg, parity):
      for p in range(NPASS):
        pltpu.sync_copy(dloc_hbm.at[p, s], dloc_v)
        nb_s, nb_d = compact(parity)
        zero_acc()
        plsc.subcore_barrier()
        sum_blocks(feats_hbm, nb_s)
        plsc.subcore_barrier()
        copy_out(out_sum, p)
        plsc.subcore_barrier()
        zero_acc()
        plsc.subcore_barrier()
        deg_blocks(nb_d)
        plsc.subcore_barrier()
        copy_out(out_deg, p)
        plsc.subcore_barrier()

    @pl.when(c == 0)
    def _():
      run_core(lo_hbm, out_lo, out_d0, 0)

    @pl.when(c == 1)
    def _():
      run_core(hi_hbm, out_hi, out_d1, 1)

  return k(feats_lo, feats_hi, src_idx, dst_loc, zeros_z, ones_b)


_TC_BLK = 400


def _tc_body(feats_ref, lo_ref, hi_ref, d0_ref, d1_ref,
             ws_ref, wn_ref, b_ref, out_ref):
  deg = d0_ref[:, 0:1] + d1_ref[:, 0:1]
  deg = jnp.maximum(deg, 1.0)
  h = jnp.concatenate([lo_ref[...], hi_ref[...]], axis=1) / deg
  acc = jnp.dot(feats_ref[...], ws_ref[...], preferred_element_type=jnp.float32)
  acc = acc + jnp.dot(h, wn_ref[...], preferred_element_type=jnp.float32)
  out_ref[...] = jnp.maximum(acc + b_ref[...], 0.0)


def _tc_combine(feats, sum_lo, sum_hi, deg0, deg1, W_self, W_neigh, bias):
  grid = (N // _TC_BLK,)
  return pl.pallas_call(
      _tc_body,
      grid=grid,
      in_specs=[
          pl.BlockSpec((_TC_BLK, D), lambda i: (i, 0)),
          pl.BlockSpec((_TC_BLK, DH), lambda i: (i, 0)),
          pl.BlockSpec((_TC_BLK, DH), lambda i: (i, 0)),
          pl.BlockSpec((_TC_BLK, DH), lambda i: (i, 0)),
          pl.BlockSpec((_TC_BLK, DH), lambda i: (i, 0)),
          pl.BlockSpec((D, D), lambda i: (0, 0)),
          pl.BlockSpec((D, D), lambda i: (0, 0)),
          pl.BlockSpec((1, D), lambda i: (0, 0)),
      ],
      out_specs=pl.BlockSpec((_TC_BLK, D), lambda i: (i, 0)),
      out_shape=jax.ShapeDtypeStruct((N, D), jnp.float32),
  )(feats, sum_lo, sum_hi, deg0, deg1, W_self, W_neigh, bias)


def kernel(feats, edge_index, W_self, W_neigh, bias):
  src = edge_index[0].astype(jnp.int32)
  dst = edge_index[1].astype(jnp.int32)
  npad = EPAD - E
  src_p = jnp.concatenate([src, jnp.zeros((npad,), jnp.int32)])
  dst_p = jnp.concatenate([dst, jnp.full((npad,), N, jnp.int32)])
  src_idx = src_p.reshape(NSUB, NB, BLK)

  locs = []
  for p in range(NPASS):
    t = dst_p - p * STRIDE
    locs.append(jnp.where((t >= 0) & (t < STRIDE), t, DUMP))
  dst_loc = jnp.stack(locs).reshape(NPASS, NSUB, NB, BLK)

  zeros_z = jnp.zeros((8, DH), jnp.float32)
  ones_b = jnp.ones((BLK, DH), jnp.float32)

  feats_lo = feats[:, :DH]
  feats_hi = feats[:, DH:]

  sum_lo, sum_hi, deg0, deg1 = _sc_aggregate(
      feats_lo, feats_hi, src_idx, dst_loc, zeros_z, ones_b)
  return _tc_combine(feats, sum_lo, sum_hi, deg0, deg1,
                     W_self, W_neigh, bias.reshape(1, D))


# speedup vs baseline: 2.5815x; 1.0363x over previous
import dataclasses
import functools

import jax
import jax.numpy as jnp
from jax import lax
from jax.experimental import pallas as pl
from jax.experimental.pallas import tpu as pltpu
from jax.experimental.pallas import tpu_sc as plsc

N = 10000
E = 160000
D = 256
DH = 128
NSUB = 16
BLK = 128
NB = 80
EPAD = NSUB * NB * BLK
SLAB = NB * BLK
CAP = SLAB + BLK

NPASS = 3
STRIDE = 3336
R_ACC = 3456
DUMP = 3400
ZR = R_ACC // NSUB
CR = 208
OUT_ROWS = NPASS * STRIDE


def _sc_aggregate(feats_lo, feats_hi, src_idx, dst_idx, zeros_z, ones_b):
  mesh = plsc.VectorSubcoreMesh(core_axis_name="c", subcore_axis_name="s")
  f32 = jnp.float32
  i32 = jnp.int32

  @functools.partial(
      pl.kernel,
      out_type=(
          jax.ShapeDtypeStruct((OUT_ROWS, DH), f32),
          jax.ShapeDtypeStruct((OUT_ROWS, DH), f32),
          jax.ShapeDtypeStruct((OUT_ROWS, DH), f32),
          jax.ShapeDtypeStruct((OUT_ROWS, DH), f32),
      ),
      mesh=mesh,
      scratch_types=[
          pltpu.VMEM((BLK, DH), f32),
          pltpu.VMEM((BLK, DH), f32),
          pltpu.VMEM((8, DH), f32),
          pltpu.VMEM((BLK, DH), f32),
          pltpu.VMEM((NB, BLK), i32),
          pltpu.VMEM((NB, BLK), i32),
          pltpu.VMEM((CAP,), i32),
          pltpu.VMEM((CAP,), i32),
          pltpu.VMEM((CAP,), i32),
          pltpu.VMEM((BLK,), i32),
          pltpu.VMEM_SHARED((R_ACC, DH), f32),
          pltpu.SemaphoreType.DMA,
          pltpu.SemaphoreType.DMA,
          pltpu.SemaphoreType.DMA,
      ],
      compiler_params=dataclasses.replace(pltpu.CompilerParams(),
                                          needs_layout_passes=False),
  )
  def k(lo_hbm, hi_hbm, src_hbm, dst_hbm, zz_hbm, ones_hbm,
        out_lo, out_hi, out_d0, out_d1,
        rows_a, rows_b, zero_v, ones_v, sidx_v, didx_v,
        csrc, cdst, cdeg, dblk_v, acc,
        sem_ga, sem_gb, sem_s):
    c = lax.axis_index("c")
    s = lax.axis_index("s")

    pltpu.sync_copy(zz_hbm, zero_v)
    pltpu.sync_copy(ones_hbm, ones_v)
    pltpu.sync_copy(src_hbm.at[s], sidx_v)
    pltpu.sync_copy(dst_hbm.at[s], didx_v)

    def zero_acc():
      @pl.loop(0, ZR, step=8)
      def _(r):
        pltpu.sync_copy(zero_v, acc.at[pl.ds(s * ZR + r, 8)])

    def copy_out(out_hbm, p):
      base = p * STRIDE
      pltpu.sync_copy(acc.at[pl.ds(s * CR, CR)],
                      out_hbm.at[pl.ds(base + s * CR, CR)])

      @pl.when(s == NSUB - 1)
      def _():
        pltpu.sync_copy(acc.at[pl.ds(NSUB * CR, STRIDE - NSUB * CR)],
                        out_hbm.at[pl.ds(base + NSUB * CR,
                                         STRIDE - NSUB * CR)])

    def compact(parity, p):
      base = p * STRIDE

      def body(j, carry):
        cs, cd = carry
        pj = (j % 2) == parity
        for rr in range(BLK // 16):
          dv = didx_v[j, pl.ds(rr * 16, 16)] - base
          sv = sidx_v[j, pl.ds(rr * 16, 16)]
          m = (dv >= 0) & (dv < STRIDE)
          plsc.store_compressed(csrc.at[pl.ds(cs, 16)], sv, mask=m)
          plsc.store_compressed(cdst.at[pl.ds(cs, 16)], dv, mask=m)
          cs = cs + jnp.sum(m.astype(i32))
          md = m & pj
          plsc.store_compressed(cdeg.at[pl.ds(cd, 16)], dv, mask=md)
          cd = cd + jnp.sum(md.astype(i32))
        return (cs, cd)

      cs, cd = lax.fori_loop(0, NB, body, (jnp.int32(0), jnp.int32(0)))
      zeros16 = jnp.zeros((16,), i32)
      dump16 = jnp.full((16,), DUMP, i32)
      for t in range(BLK // 16):
        csrc[pl.ds(cs + t * 16, 16)] = zeros16
        cdst[pl.ds(cs + t * 16, 16)] = dump16
        cdeg[pl.ds(cd + t * 16, 16)] = dump16
      return (cs + 127) // 128, (cd + 127) // 128

    def sum_blocks(feats_hbm, nb_s):
      @pl.when(nb_s > 0)
      def _():
        pltpu.async_copy(feats_hbm.at[csrc.at[pl.ds(0, BLK)]],
                         rows_a, sem_ga)

      @pl.when(nb_s > 1)
      def _():
        pltpu.async_copy(feats_hbm.at[csrc.at[pl.ds(BLK, BLK)]],
                         rows_b, sem_gb)

      def body(bb, _):
        for (buf, sem_g, off) in ((rows_a, sem_ga, 0), (rows_b, sem_gb, 1)):
          b = bb * 2 + off

          @pl.when(b < nb_s)
          def _():
            pltpu.make_async_copy(
                feats_hbm.at[csrc.at[pl.ds(b * BLK, BLK)]],
                buf, sem_g).wait()
            for rr in range(BLK // 16):
              dblk_v[pl.ds(rr * 16, 16)] = cdst[pl.ds(b * BLK + rr * 16, 16)]
            pltpu.sync_copy(buf, acc.at[dblk_v], add=True)

            @pl.when(b + 2 < nb_s)
            def _():
              pltpu.async_copy(
                  feats_hbm.at[csrc.at[pl.ds((b + 2) * BLK, BLK)]],
                  buf, sem_g)
        return 0

      lax.fori_loop(0, (nb_s + 1) // 2, body, 0)

    def deg_blocks(nb_d):
      def body(b, _):
        for rr in range(BLK // 16):
          dblk_v[pl.ds(rr * 16, 16)] = cdeg[pl.ds(b * BLK + rr * 16, 16)]
        pltpu.sync_copy(ones_v, acc.at[dblk_v], add=True)
        return 0

      lax.fori_loop(0, nb_d, body, 0)

    def run_core(feats_hbm, out_sum, out_deg, parity):
      for p in range(NPASS):
        nb_s, nb_d = compact(parity, p)
        zero_acc()
        plsc.subcore_barrier()
        sum_blocks(feats_hbm, nb_s)
        plsc.subcore_barrier()
        copy_out(out_sum, p)
        plsc.subcore_barrier()
        zero_acc()
        plsc.subcore_barrier()
        deg_blocks(nb_d)
        plsc.subcore_barrier()
        copy_out(out_deg, p)
        plsc.subcore_barrier()

    @pl.when(c == 0)
    def _():
      run_core(lo_hbm, out_lo, out_d0, 0)

    @pl.when(c == 1)
    def _():
      run_core(hi_hbm, out_hi, out_d1, 1)

  return k(feats_lo, feats_hi, src_idx, dst_idx, zeros_z, ones_b)


_TC_BLK = 400


def _tc_body(feats_ref, lo_ref, hi_ref, d0_ref, d1_ref,
             ws_ref, wn_ref, b_ref, out_ref):
  deg = d0_ref[:, 0:1] + d1_ref[:, 0:1]
  deg = jnp.maximum(deg, 1.0)
  h = jnp.concatenate([lo_ref[...], hi_ref[...]], axis=1) / deg
  acc = jnp.dot(feats_ref[...], ws_ref[...], preferred_element_type=jnp.float32)
  acc = acc + jnp.dot(h, wn_ref[...], preferred_element_type=jnp.float32)
  out_ref[...] = jnp.maximum(acc + b_ref[...], 0.0)


def _tc_combine(feats, sum_lo, sum_hi, deg0, deg1, W_self, W_neigh, bias):
  grid = (N // _TC_BLK,)
  return pl.pallas_call(
      _tc_body,
      grid=grid,
      in_specs=[
          pl.BlockSpec((_TC_BLK, D), lambda i: (i, 0)),
          pl.BlockSpec((_TC_BLK, DH), lambda i: (i, 0)),
          pl.BlockSpec((_TC_BLK, DH), lambda i: (i, 0)),
          pl.BlockSpec((_TC_BLK, DH), lambda i: (i, 0)),
          pl.BlockSpec((_TC_BLK, DH), lambda i: (i, 0)),
          pl.BlockSpec((D, D), lambda i: (0, 0)),
          pl.BlockSpec((D, D), lambda i: (0, 0)),
          pl.BlockSpec((1, D), lambda i: (0, 0)),
      ],
      out_specs=pl.BlockSpec((_TC_BLK, D), lambda i: (i, 0)),
      out_shape=jax.ShapeDtypeStruct((N, D), jnp.float32),
  )(feats, sum_lo, sum_hi, deg0, deg1, W_self, W_neigh, bias)


def kernel(feats, edge_index, W_self, W_neigh, bias):
  src = edge_index[0].astype(jnp.int32)
  dst = edge_index[1].astype(jnp.int32)
  npad = EPAD - E
  src_p = jnp.concatenate([src, jnp.zeros((npad,), jnp.int32)])
  dst_p = jnp.concatenate([dst, jnp.full((npad,), N, jnp.int32)])
  src_idx = src_p.reshape(NSUB, NB, BLK)
  dst_idx = dst_p.reshape(NSUB, NB, BLK)

  zeros_z = jnp.zeros((8, DH), jnp.float32)
  ones_b = jnp.ones((BLK, DH), jnp.float32)

  feats_lo = feats[:, :DH]
  feats_hi = feats[:, DH:]

  sum_lo, sum_hi, deg0, deg1 = _sc_aggregate(
      feats_lo, feats_hi, src_idx, dst_idx, zeros_z, ones_b)
  return _tc_combine(feats, sum_lo, sum_hi, deg0, deg1,
                     W_self, W_neigh, bias.reshape(1, D))


# speedup vs baseline: 2.7309x; 1.0579x over previous
import dataclasses
import functools

import jax
import jax.numpy as jnp
from jax import lax
from jax.experimental import pallas as pl
from jax.experimental.pallas import tpu as pltpu
from jax.experimental.pallas import tpu_sc as plsc

N = 10000
E = 160000
D = 256
DH = 128
NSUB = 16
BLK = 128
NB = 80
EPAD = NSUB * NB * BLK
SLAB = NB * BLK
CAP = SLAB + BLK

NPASS = 3
STRIDE = 3336
R_ACC = 3456
DUMP = 3400
ZR = R_ACC // NSUB
CR = 208
OUT_ROWS = 10240
NHIST = 10240
NW = 2 * NSUB


def _sc_aggregate(feats_lo, feats_hi, src_idx, dst_idx, zeros_z):
  mesh = plsc.VectorSubcoreMesh(core_axis_name="c", subcore_axis_name="s")
  f32 = jnp.float32
  i32 = jnp.int32

  @functools.partial(
      pl.kernel,
      out_type=(
          jax.ShapeDtypeStruct((OUT_ROWS, DH), f32),
          jax.ShapeDtypeStruct((OUT_ROWS, DH), f32),
          jax.ShapeDtypeStruct((NW, 1, NHIST), f32),
      ),
      mesh=mesh,
      scratch_types=[
          pltpu.VMEM((BLK, DH), f32),
          pltpu.VMEM((BLK, DH), f32),
          pltpu.VMEM((8, DH), f32),
          pltpu.VMEM((NB, BLK), i32),
          pltpu.VMEM((NB, BLK), i32),
          pltpu.VMEM((CAP,), i32),
          pltpu.VMEM((CAP,), i32),
          pltpu.VMEM((NHIST,), f32),
          pltpu.VMEM((BLK,), i32),
          pltpu.VMEM_SHARED((R_ACC, DH), f32),
          pltpu.SemaphoreType.DMA,
          pltpu.SemaphoreType.DMA,
          pltpu.SemaphoreType.DMA,
      ],
      compiler_params=dataclasses.replace(pltpu.CompilerParams(),
                                          needs_layout_passes=False),
  )
  def k(lo_hbm, hi_hbm, src_hbm, dst_hbm, zz_hbm,
        out_lo, out_hi, out_deg,
        rows_a, rows_b, zero_v, sidx_v, didx_v,
        csrc, cdst, hist_v, dblk_v, acc,
        sem_ga, sem_gb, sem_s):
    c = lax.axis_index("c")
    s = lax.axis_index("s")

    pltpu.sync_copy(zz_hbm, zero_v)
    pltpu.sync_copy(src_hbm.at[s], sidx_v)
    pltpu.sync_copy(dst_hbm.at[s], didx_v)

    def degree_hist(parity):
      zeros16 = jnp.zeros((16,), f32)

      @pl.loop(0, NHIST, step=16)
      def _(r):
        hist_v[pl.ds(r, 16)] = zeros16

      ones16 = jnp.ones((16,), f32)

      def body(j, carry):
        for rr in range(BLK // 16):
          dv = didx_v[j, pl.ds(rr * 16, 16)]
          plsc.addupdate_scatter(hist_v, [dv], ones16)
        return carry

      lax.fori_loop(0, NB // 2, lambda jj, cy: body(2 * jj + parity, cy), 0)
      pltpu.sync_copy(hist_v, out_deg.at[c * NSUB + s, 0])

    def zero_acc():
      @pl.loop(0, ZR, step=8)
      def _(r):
        pltpu.sync_copy(zero_v, acc.at[pl.ds(s * ZR + r, 8)])

    def copy_out(out_hbm, p):
      base = p * STRIDE
      pltpu.sync_copy(acc.at[pl.ds(s * CR, CR)],
                      out_hbm.at[pl.ds(base + s * CR, CR)])

      @pl.when(s == NSUB - 1)
      def _():
        pltpu.sync_copy(acc.at[pl.ds(NSUB * CR, STRIDE - NSUB * CR)],
                        out_hbm.at[pl.ds(base + NSUB * CR,
                                         STRIDE - NSUB * CR)])

    def compact(p):
      base = p * STRIDE

      def body(j, cs):
        for rr in range(BLK // 16):
          dv = didx_v[j, pl.ds(rr * 16, 16)] - base
          sv = sidx_v[j, pl.ds(rr * 16, 16)]
          m = (dv >= 0) & (dv < STRIDE)
          plsc.store_compressed(csrc.at[pl.ds(cs, 16)], sv, mask=m)
          plsc.store_compressed(cdst.at[pl.ds(cs, 16)], dv, mask=m)
          cs = cs + jnp.sum(m.astype(i32))
        return cs

      cs = lax.fori_loop(0, NB, body, jnp.int32(0))
      zeros16 = jnp.zeros((16,), i32)
      dump16 = jnp.full((16,), DUMP, i32)
      for t in range(BLK // 16):
        csrc[pl.ds(cs + t * 16, 16)] = zeros16
        cdst[pl.ds(cs + t * 16, 16)] = dump16
      return (cs + 127) // 128

    def sum_blocks(feats_hbm, nb_s):
      @pl.when(nb_s > 0)
      def _():
        pltpu.async_copy(feats_hbm.at[csrc.at[pl.ds(0, BLK)]],
                         rows_a, sem_ga)

      @pl.when(nb_s > 1)
      def _():
        pltpu.async_copy(feats_hbm.at[csrc.at[pl.ds(BLK, BLK)]],
                         rows_b, sem_gb)

      def body(bb, _):
        for (buf, sem_g, off) in ((rows_a, sem_ga, 0), (rows_b, sem_gb, 1)):
          b = bb * 2 + off

          @pl.when(b < nb_s)
          def _():
            pltpu.make_async_copy(
                feats_hbm.at[csrc.at[pl.ds(b * BLK, BLK)]],
                buf, sem_g).wait()
            for rr in range(BLK // 16):
              dblk_v[pl.ds(rr * 16, 16)] = cdst[pl.ds(b * BLK + rr * 16, 16)]
            pltpu.sync_copy(buf, acc.at[dblk_v], add=True)

            @pl.when(b + 2 < nb_s)
            def _():
              pltpu.async_copy(
                  feats_hbm.at[csrc.at[pl.ds((b + 2) * BLK, BLK)]],
                  buf, sem_g)
        return 0

      lax.fori_loop(0, (nb_s + 1) // 2, body, 0)

    def run_core(feats_hbm, out_sum, parity):
      degree_hist(parity)
      for p in range(NPASS):
        nb_s = compact(p)
        zero_acc()
        plsc.subcore_barrier()
        sum_blocks(feats_hbm, nb_s)
        plsc.subcore_barrier()
        copy_out(out_sum, p)
        plsc.subcore_barrier()

    @pl.when(c == 0)
    def _():
      run_core(lo_hbm, out_lo, 0)

    @pl.when(c == 1)
    def _():
      run_core(hi_hbm, out_hi, 1)

  return k(feats_lo, feats_hi, src_idx, dst_idx, zeros_z)


_TC_BLK = 512
_TC_ROWS = 10240


def _tc_body(feats_ref, lo_ref, hi_ref, dh_ref,
             ws_ref, wn_ref, b_ref, out_ref):
  i = pl.program_id(0)
  dh = dh_ref[:, 0, pl.ds(i * _TC_BLK, _TC_BLK)]
  deg = jnp.maximum(jnp.sum(dh, axis=0), 1.0)[:, None]
  h = jnp.concatenate([lo_ref[...], hi_ref[...]], axis=1) / deg
  acc = jnp.dot(feats_ref[...], ws_ref[...], preferred_element_type=jnp.float32)
  acc = acc + jnp.dot(h, wn_ref[...], preferred_element_type=jnp.float32)
  out_ref[...] = jnp.maximum(acc + b_ref[...], 0.0)


def _tc_combine(feats, sum_lo, sum_hi, deg_h, W_self, W_neigh, bias):
  grid = (_TC_ROWS // _TC_BLK,)
  return pl.pallas_call(
      _tc_body,
      grid=grid,
      in_specs=[
          pl.BlockSpec((_TC_BLK, D), lambda i: (i, 0)),
          pl.BlockSpec((_TC_BLK, DH), lambda i: (i, 0)),
          pl.BlockSpec((_TC_BLK, DH), lambda i: (i, 0)),
          pl.BlockSpec((NW, 1, NHIST), lambda i: (0, 0, 0)),
          pl.BlockSpec((D, D), lambda i: (0, 0)),
          pl.BlockSpec((D, D), lambda i: (0, 0)),
          pl.BlockSpec((1, D), lambda i: (0, 0)),
      ],
      out_specs=pl.BlockSpec((_TC_BLK, D), lambda i: (i, 0)),
      out_shape=jax.ShapeDtypeStruct((_TC_ROWS, D), jnp.float32),
  )(feats, sum_lo, sum_hi, deg_h, W_self, W_neigh, bias)


def kernel(feats, edge_index, W_self, W_neigh, bias):
  src = edge_index[0].astype(jnp.int32)
  dst = edge_index[1].astype(jnp.int32)
  npad = EPAD - E
  src_p = jnp.concatenate([src, jnp.zeros((npad,), jnp.int32)])
  dst_p = jnp.concatenate([dst, jnp.full((npad,), N, jnp.int32)])
  src_idx = src_p.reshape(NSUB, NB, BLK)
  dst_idx = dst_p.reshape(NSUB, NB, BLK)

  zeros_z = jnp.zeros((8, DH), jnp.float32)

  feats_p = jnp.pad(feats, ((0, _TC_ROWS - N), (0, 0)))
  feats_lo = feats_p[:, :DH]
  feats_hi = feats_p[:, DH:]

  sum_lo, sum_hi, deg_h = _sc_aggregate(
      feats_lo, feats_hi, src_idx, dst_idx, zeros_z)
  out = _tc_combine(feats_p, sum_lo, sum_hi, deg_h,
                    W_self, W_neigh, bias.reshape(1, D))
  return out[:N]


# speedup vs baseline: 2.7772x; 1.0169x over previous
import dataclasses
import functools

import jax
import jax.numpy as jnp
from jax import lax
from jax.experimental import pallas as pl
from jax.experimental.pallas import tpu as pltpu
from jax.experimental.pallas import tpu_sc as plsc

N = 10000
E = 160000
D = 256
DH = 128
NSUB = 16
BLK = 128
NB = 80
EPAD = NSUB * NB * BLK
SLAB = NB * BLK
CAP = SLAB + BLK

NPASS = 3
STRIDE = 3336
R_ACC = 3456
DUMP = 3400
ZR = R_ACC // NSUB
CR = 208
OUT_ROWS = 10240
NHIST = 10240
NW = 2 * NSUB


def _sc_aggregate(feats_lo, feats_hi, src_idx, dst_idx, zeros_z):
  mesh = plsc.VectorSubcoreMesh(core_axis_name="c", subcore_axis_name="s")
  f32 = jnp.float32
  i32 = jnp.int32

  @functools.partial(
      pl.kernel,
      out_type=(
          jax.ShapeDtypeStruct((OUT_ROWS, DH), f32),
          jax.ShapeDtypeStruct((OUT_ROWS, DH), f32),
          jax.ShapeDtypeStruct((NW, 1, NHIST), f32),
      ),
      mesh=mesh,
      scratch_types=[
          pltpu.VMEM((BLK, DH), f32),
          pltpu.VMEM((BLK, DH), f32),
          pltpu.VMEM((BLK, DH), f32),
          pltpu.VMEM((8, DH), f32),
          pltpu.VMEM((NB, BLK), i32),
          pltpu.VMEM((NB, BLK), i32),
          pltpu.VMEM((CAP,), i32),
          pltpu.VMEM((CAP,), i32),
          pltpu.VMEM((NHIST,), f32),
          pltpu.VMEM((BLK,), i32),
          pltpu.VMEM_SHARED((R_ACC, DH), f32),
          pltpu.SemaphoreType.DMA,
          pltpu.SemaphoreType.DMA,
          pltpu.SemaphoreType.DMA,
      ],
      compiler_params=dataclasses.replace(pltpu.CompilerParams(),
                                          needs_layout_passes=False),
  )
  def k(lo_hbm, hi_hbm, src_hbm, dst_hbm, zz_hbm,
        out_lo, out_hi, out_deg,
        rows_a, rows_b, rows_c, zero_v, sidx_v, didx_v,
        csrc, cdst, hist_v, dblk_v, acc,
        sem_ga, sem_gb, sem_gc):
    c = lax.axis_index("c")
    s = lax.axis_index("s")

    pltpu.sync_copy(zz_hbm, zero_v)
    pltpu.sync_copy(src_hbm.at[s], sidx_v)
    pltpu.sync_copy(dst_hbm.at[s], didx_v)

    def degree_hist(parity):
      zeros16 = jnp.zeros((16,), f32)

      @pl.loop(0, NHIST, step=16)
      def _(r):
        hist_v[pl.ds(r, 16)] = zeros16

      ones16 = jnp.ones((16,), f32)

      def body(j, carry):
        for rr in range(BLK // 16):
          dv = didx_v[j, pl.ds(rr * 16, 16)]
          plsc.addupdate_scatter(hist_v, [dv], ones16)
        return carry

      lax.fori_loop(0, NB // 2, lambda jj, cy: body(2 * jj + parity, cy), 0)
      pltpu.sync_copy(hist_v, out_deg.at[c * NSUB + s, 0])

    def zero_acc():
      @pl.loop(0, ZR, step=8)
      def _(r):
        pltpu.sync_copy(zero_v, acc.at[pl.ds(s * ZR + r, 8)])

    def copy_out(out_hbm, p):
      base = p * STRIDE
      pltpu.sync_copy(acc.at[pl.ds(s * CR, CR)],
                      out_hbm.at[pl.ds(base + s * CR, CR)])

      @pl.when(s == NSUB - 1)
      def _():
        pltpu.sync_copy(acc.at[pl.ds(NSUB * CR, STRIDE - NSUB * CR)],
                        out_hbm.at[pl.ds(base + NSUB * CR,
                                         STRIDE - NSUB * CR)])

    def compact(p):
      base = p * STRIDE

      def body(j, cs):
        for rr in range(BLK // 16):
          dv = didx_v[j, pl.ds(rr * 16, 16)] - base
          sv = sidx_v[j, pl.ds(rr * 16, 16)]
          m = (dv >= 0) & (dv < STRIDE)
          plsc.store_compressed(csrc.at[pl.ds(cs, 16)], sv, mask=m)
          plsc.store_compressed(cdst.at[pl.ds(cs, 16)], dv, mask=m)
          cs = cs + jnp.sum(m.astype(i32))
        return cs

      cs = lax.fori_loop(0, NB, body, jnp.int32(0))
      zeros16 = jnp.zeros((16,), i32)
      dump16 = jnp.full((16,), DUMP, i32)
      for t in range(BLK // 16):
        csrc[pl.ds(cs + t * 16, 16)] = zeros16
        cdst[pl.ds(cs + t * 16, 16)] = dump16
      return (cs + 127) // 128

    def sum_blocks(feats_hbm, nb_s):
      slots = ((rows_a, sem_ga), (rows_b, sem_gb), (rows_c, sem_gc))
      nbuf = len(slots)
      for t, (buf, sem_g) in enumerate(slots):
        @pl.when(t < nb_s)
        def _():
          pltpu.async_copy(feats_hbm.at[csrc.at[pl.ds(t * BLK, BLK)]],
                           buf, sem_g)

      def body(bb, _):
        for off, (buf, sem_g) in enumerate(slots):
          b = bb * nbuf + off

          @pl.when(b < nb_s)
          def _():
            pltpu.make_async_copy(
                feats_hbm.at[csrc.at[pl.ds(b * BLK, BLK)]],
                buf, sem_g).wait()
            for rr in range(BLK // 16):
              dblk_v[pl.ds(rr * 16, 16)] = cdst[pl.ds(b * BLK + rr * 16, 16)]
            pltpu.sync_copy(buf, acc.at[dblk_v], add=True)

            @pl.when(b + nbuf < nb_s)
            def _():
              pltpu.async_copy(
                  feats_hbm.at[csrc.at[pl.ds((b + nbuf) * BLK, BLK)]],
                  buf, sem_g)
        return 0

      lax.fori_loop(0, (nb_s + nbuf - 1) // nbuf, body, 0)

    def run_core(feats_hbm, out_sum, parity):
      degree_hist(parity)
      for p in range(NPASS):
        nb_s = compact(p)
        zero_acc()
        plsc.subcore_barrier()
        sum_blocks(feats_hbm, nb_s)
        plsc.subcore_barrier()
        copy_out(out_sum, p)
        plsc.subcore_barrier()

    @pl.when(c == 0)
    def _():
      run_core(lo_hbm, out_lo, 0)

    @pl.when(c == 1)
    def _():
      run_core(hi_hbm, out_hi, 1)

  return k(feats_lo, feats_hi, src_idx, dst_idx, zeros_z)


_TC_BLK = 512
_TC_ROWS = 10240


def _tc_body(feats_ref, lo_ref, hi_ref, dh_ref,
             ws_ref, wn_ref, b_ref, out_ref):
  i = pl.program_id(0)
  dh = dh_ref[:, 0, pl.ds(i * _TC_BLK, _TC_BLK)]
  deg = jnp.maximum(jnp.sum(dh, axis=0), 1.0)[:, None]
  h = jnp.concatenate([lo_ref[...], hi_ref[...]], axis=1) / deg
  acc = jnp.dot(feats_ref[...], ws_ref[...], preferred_element_type=jnp.float32)
  acc = acc + jnp.dot(h, wn_ref[...], preferred_element_type=jnp.float32)
  out_ref[...] = jnp.maximum(acc + b_ref[...], 0.0)


def _tc_combine(feats, sum_lo, sum_hi, deg_h, W_self, W_neigh, bias):
  grid = (_TC_ROWS // _TC_BLK,)
  return pl.pallas_call(
      _tc_body,
      grid=grid,
      in_specs=[
          pl.BlockSpec((_TC_BLK, D), lambda i: (i, 0)),
          pl.BlockSpec((_TC_BLK, DH), lambda i: (i, 0)),
          pl.BlockSpec((_TC_BLK, DH), lambda i: (i, 0)),
          pl.BlockSpec((NW, 1, NHIST), lambda i: (0, 0, 0)),
          pl.BlockSpec((D, D), lambda i: (0, 0)),
          pl.BlockSpec((D, D), lambda i: (0, 0)),
          pl.BlockSpec((1, D), lambda i: (0, 0)),
      ],
      out_specs=pl.BlockSpec((_TC_BLK, D), lambda i: (i, 0)),
      out_shape=jax.ShapeDtypeStruct((_TC_ROWS, D), jnp.float32),
  )(feats, sum_lo, sum_hi, deg_h, W_self, W_neigh, bias)


def kernel(feats, edge_index, W_self, W_neigh, bias):
  src = edge_index[0].astype(jnp.int32)
  dst = edge_index[1].astype(jnp.int32)
  npad = EPAD - E
  src_p = jnp.concatenate([src, jnp.zeros((npad,), jnp.int32)])
  dst_p = jnp.concatenate([dst, jnp.full((npad,), N, jnp.int32)])
  src_idx = src_p.reshape(NSUB, NB, BLK)
  dst_idx = dst_p.reshape(NSUB, NB, BLK)

  zeros_z = jnp.zeros((8, DH), jnp.float32)

  feats_p = jnp.pad(feats, ((0, _TC_ROWS - N), (0, 0)))
  feats_lo = feats_p[:, :DH]
  feats_hi = feats_p[:, DH:]

  sum_lo, sum_hi, deg_h = _sc_aggregate(
      feats_lo, feats_hi, src_idx, dst_idx, zeros_z)
  out = _tc_combine(feats_p, sum_lo, sum_hi, deg_h,
                    W_self, W_neigh, bias.reshape(1, D))
  return out[:N]
